# baseline (speedup 1.0000x reference)
import functools

import jax
import jax.numpy as jnp
from jax.experimental import pallas as pl
from jax.experimental.pallas import tpu as pltpu

N = 16384
F = 128
K = 1000
BN = 512


def _kmeans_block(x_ref, ct_ref, dist_ref, asn_ref, inr_ref):
    x = x_ref[...]
    ct = ct_ref[...]
    x_sq = jnp.sum(x * x, axis=1, keepdims=True)
    c_sq = jnp.sum(ct * ct, axis=0, keepdims=True)
    cross = jnp.dot(x, ct, preferred_element_type=jnp.float32)
    d2 = jnp.maximum(x_sq + c_sq - 2.0 * cross, 0.0)
    dist = jnp.sqrt(d2)
    dist_ref[...] = dist
    m = jnp.min(dist, axis=1, keepdims=True)
    idx = jax.lax.broadcasted_iota(jnp.int32, dist.shape, 1)
    first_min = jnp.min(jnp.where(dist == m, idx, K), axis=1)
    asn_ref[...] = first_min
    inr_ref[...] = (m * m)[:, 0]


@jax.jit
def kernel(data, centroids):
    ct = centroids.T
    grid = (N // BN,)
    distances, assignments, inertias = pl.pallas_call(
        _kmeans_block,
        grid=grid,
        in_specs=[
            pl.BlockSpec((BN, F), lambda i: (i, 0)),
            pl.BlockSpec((F, K), lambda i: (0, 0)),
        ],
        out_specs=[
            pl.BlockSpec((BN, K), lambda i: (i, 0)),
            pl.BlockSpec((BN,), lambda i: (i,)),
            pl.BlockSpec((BN,), lambda i: (i,)),
        ],
        out_shape=[
            jax.ShapeDtypeStruct((N, K), jnp.float32),
            jax.ShapeDtypeStruct((N,), jnp.int32),
            jax.ShapeDtypeStruct((N,), jnp.float32),
        ],
        compiler_params=pltpu.CompilerParams(
            dimension_semantics=("arbitrary",),
        ),
    )(data, ct)
    return (distances, assignments, inertias)
